# Initial kernel scaffold; baseline (speedup 1.0000x reference)
import jax
import jax.numpy as jnp
from jax.experimental import pallas as pl


def kernel(x_nchw, weight_oihw, bias):
    raise NotImplementedError("write your pallas kernel here")


# speedup vs baseline: 1.0032x; 1.0032x over previous
import functools

import jax
import jax.numpy as jnp
from jax import lax
from jax.experimental import pallas as pl
from jax.experimental.pallas import tpu as pltpu


def _conv_sig_kernel(x_ref, w_ref, b_ref, o_ref, y_ref, *, cin, cout, k, wp,
                     ho, wo):
    L = ho * wp

    xb = x_ref[0].astype(jnp.bfloat16)
    parts = [xb[:, kh * wp + kw:kh * wp + kw + L]
             for kh in range(k) for kw in range(k)]
    xcat = jnp.concatenate(parts, axis=0)

    acc = jnp.dot(w_ref[...], xcat,
                  preferred_element_type=jnp.float32)
    acc = acc + b_ref[...]

    d = 1.0 + jnp.exp(-acc)
    r = pl.reciprocal(d, approx=True)
    y_ref[...] = r * (2.0 - d * r)

    for h in range(ho):
        o_ref[0, :, h, :] = y_ref[:, pl.ds(h * wp, wo)]


def kernel(x_nchw, weight_oihw, bias):
    pad = 1
    N, Cin, H, W = x_nchw.shape
    Cout, Cin_w, K, K2 = weight_oihw.shape
    assert Cin == Cin_w and K == K2

    Ho = H + 2 * pad - K + 1
    Wo = W + 2 * pad - K + 1
    Hp = H + 2 * pad
    Wp = W + 2 * pad

    x_pad = jnp.pad(x_nchw, ((0, 0), (0, 0), (pad, pad + 1), (pad, pad)))
    Lp = (Hp + 1) * Wp
    x_flat = x_pad.reshape(N, Cin, Lp)

    w_all = jnp.transpose(weight_oihw, (0, 2, 3, 1)).reshape(Cout, K * K * Cin)
    w_all = w_all.astype(jnp.bfloat16)
    b2d = bias.astype(jnp.float32).reshape(Cout, 1)

    kernel_fn = functools.partial(_conv_sig_kernel, cin=Cin, cout=Cout, k=K,
                                  wp=Wp, ho=Ho, wo=Wo)

    out = pl.pallas_call(
        kernel_fn,
        out_shape=jax.ShapeDtypeStruct((N, Cout, Ho, Wo), x_nchw.dtype),
        grid=(N,),
        in_specs=[
            pl.BlockSpec((1, Cin, Lp), lambda n: (n, 0, 0)),
            pl.BlockSpec((Cout, K * K * Cin), lambda n: (0, 0)),
            pl.BlockSpec((Cout, 1), lambda n: (0, 0)),
        ],
        out_specs=pl.BlockSpec((1, Cout, Ho, Wo), lambda n: (n, 0, 0, 0)),
        scratch_shapes=[pltpu.VMEM((Cout, Ho * Wp), jnp.float32)],
        compiler_params=pltpu.CompilerParams(
            dimension_semantics=("parallel",),
            vmem_limit_bytes=64 * 1024 * 1024,
        ),
    )(x_flat, w_all, b2d)
    return out


# speedup vs baseline: 1.7573x; 1.7517x over previous
import functools

import jax
import jax.numpy as jnp
from jax import lax
from jax.experimental import pallas as pl
from jax.experimental.pallas import tpu as pltpu


def _conv_sig_kernel(x_ref, w_ref, b_ref, o_ref, *, cin, cout, k, wp,
                     ho, wo):
    L = ho * wp

    xb = x_ref[0].astype(jnp.bfloat16)
    parts = [xb[:, kh * wp + kw:kh * wp + kw + L]
             for kh in range(k) for kw in range(k)]
    xcat = jnp.concatenate(parts, axis=0)

    acc = jnp.dot(w_ref[...], xcat,
                  preferred_element_type=jnp.float32)
    acc = acc + b_ref[...]

    d = 1.0 + jnp.exp(-acc)
    r = pl.reciprocal(d, approx=True)
    y = r * (2.0 - d * r)

    for h in range(ho):
        o_ref[0, :, h * wo:(h + 1) * wo] = y[:, h * wp:h * wp + wo]


def kernel(x_nchw, weight_oihw, bias):
    pad = 1
    N, Cin, H, W = x_nchw.shape
    Cout, Cin_w, K, K2 = weight_oihw.shape
    assert Cin == Cin_w and K == K2

    Ho = H + 2 * pad - K + 1
    Wo = W + 2 * pad - K + 1
    Hp = H + 2 * pad
    Wp = W + 2 * pad

    x_pad = jnp.pad(x_nchw, ((0, 0), (0, 0), (pad, pad + 1), (pad, pad)))
    Lp = (Hp + 1) * Wp
    x_flat = x_pad.reshape(N, Cin, Lp)

    w_all = jnp.transpose(weight_oihw, (0, 2, 3, 1)).reshape(Cout, K * K * Cin)
    w_all = w_all.astype(jnp.bfloat16)
    b2d = bias.astype(jnp.float32).reshape(Cout, 1)

    kernel_fn = functools.partial(_conv_sig_kernel, cin=Cin, cout=Cout, k=K,
                                  wp=Wp, ho=Ho, wo=Wo)

    out = pl.pallas_call(
        kernel_fn,
        out_shape=jax.ShapeDtypeStruct((N, Cout, Ho * Wo), x_nchw.dtype),
        grid=(N,),
        in_specs=[
            pl.BlockSpec((1, Cin, Lp), lambda n: (n, 0, 0)),
            pl.BlockSpec((Cout, K * K * Cin), lambda n: (0, 0)),
            pl.BlockSpec((Cout, 1), lambda n: (0, 0)),
        ],
        out_specs=pl.BlockSpec((1, Cout, Ho * Wo), lambda n: (n, 0, 0)),
        compiler_params=pltpu.CompilerParams(
            dimension_semantics=("parallel",),
            vmem_limit_bytes=64 * 1024 * 1024,
        ),
    )(x_flat, w_all, b2d)
    return out.reshape(N, Cout, Ho, Wo)


# speedup vs baseline: 5.2534x; 2.9896x over previous
import functools

import jax
import jax.numpy as jnp
from jax.experimental import pallas as pl
from jax.experimental.pallas import tpu as pltpu


def _conv_sig_kernel(x_ref, w_ref, b_ref, m_ref, o_ref, s_ref, *, cin, cout,
                     k, ho, wo, r0, rows):
    L = ho * wo
    LX = (k - 1) * wo + L

    s_ref[:r0, :] = jnp.zeros((r0, cin), jnp.bfloat16)
    s_ref[r0 + L:, :] = jnp.zeros((rows - r0 - L, cin), jnp.bfloat16)
    s_ref[r0:r0 + L, :] = x_ref[0].astype(jnp.bfloat16)

    b0 = r0 - wo - 1
    p0 = s_ref[b0:b0 + LX, :] * m_ref[0, b0:b0 + LX, :]
    p1 = s_ref[b0 + 1:b0 + 1 + LX, :]
    p2 = s_ref[b0 + 2:b0 + 2 + LX, :] * m_ref[1, b0 + 2:b0 + 2 + LX, :]
    x3 = jnp.concatenate([p0, p1, p2], axis=1)

    u0 = jnp.dot(x3, w_ref[0], preferred_element_type=jnp.float32)
    u1 = jnp.dot(x3, w_ref[1], preferred_element_type=jnp.float32)
    u2 = jnp.dot(x3, w_ref[2], preferred_element_type=jnp.float32)
    acc = (u0[0:L] + u1[wo:wo + L] + u2[2 * wo:2 * wo + L]
           + b_ref[...].astype(jnp.float32))

    d = 1.0 + jnp.exp(-acc)
    r = pl.reciprocal(d, approx=True)
    o_ref[0] = r * (2.0 - d * r)


def kernel(x_nchw, weight_oihw, bias):
    N, Cin, H, W = x_nchw.shape
    Cout, Cin_w, K, K2 = weight_oihw.shape
    assert Cin == Cin_w and K == K2

    Ho, Wo = H, W
    L = Ho * Wo
    R0 = 64
    ROWS = ((R0 + L + Wo + 2 + 7) // 8 + 1) * 8
    ROWS = ((ROWS + 15) // 16) * 16

    x_pix = jnp.transpose(x_nchw, (0, 2, 3, 1)).reshape(N, L, Cin)

    w3 = jnp.transpose(weight_oihw, (2, 3, 1, 0)).reshape(K, K * Cin, Cout)
    w3 = w3.astype(jnp.bfloat16)
    b2d = bias.astype(jnp.float32).reshape(1, Cout)

    r = jnp.arange(ROWS)
    q = r - R0
    in_data = (q >= 0) & (q < L)
    col = q % Wo
    m0 = jnp.where(in_data & (col == Wo - 1), 0.0, 1.0)
    m2 = jnp.where(in_data & (col == 0), 0.0, 1.0)
    masks = jnp.stack([m0, m2]).astype(jnp.bfloat16)
    masks = jnp.broadcast_to(masks[:, :, None], (2, ROWS, Cin))

    kernel_fn = functools.partial(_conv_sig_kernel, cin=Cin, cout=Cout, k=K,
                                  ho=Ho, wo=Wo, r0=R0, rows=ROWS)

    out = pl.pallas_call(
        kernel_fn,
        out_shape=jax.ShapeDtypeStruct((N, L, Cout), x_nchw.dtype),
        grid=(N,),
        in_specs=[
            pl.BlockSpec((1, L, Cin), lambda n: (n, 0, 0)),
            pl.BlockSpec((K, K * Cin, Cout), lambda n: (0, 0, 0)),
            pl.BlockSpec((1, Cout), lambda n: (0, 0)),
            pl.BlockSpec((2, ROWS, Cin), lambda n: (0, 0, 0)),
        ],
        out_specs=pl.BlockSpec((1, L, Cout), lambda n: (n, 0, 0)),
        scratch_shapes=[pltpu.VMEM((ROWS, Cin), jnp.bfloat16)],
        compiler_params=pltpu.CompilerParams(
            dimension_semantics=("parallel",),
            vmem_limit_bytes=64 * 1024 * 1024,
        ),
    )(x_pix, w3, b2d, masks)
    return jnp.transpose(out.reshape(N, Ho, Wo, Cout), (0, 3, 1, 2))


# speedup vs baseline: 5.5313x; 1.0529x over previous
import functools

import jax
import jax.numpy as jnp
import numpy as np
from jax.experimental import pallas as pl
from jax.experimental.pallas import tpu as pltpu


def _conv_sig_kernel(x_ref, w_ref, b_ref, m_ref, o_ref, s_ref, *, cin, cout,
                     k, ho, wo, r0, rows, ipb):
    L = ho * wo
    LX = (k - 1) * wo + L
    b0 = r0 - wo - 1

    chunks = []
    for j in range(ipb):
        s = s_ref.at[j]
        s[:r0, :] = jnp.zeros((r0, cin), jnp.bfloat16)
        s[r0 + L:, :] = jnp.zeros((rows - r0 - L, cin), jnp.bfloat16)
        s[r0:r0 + L, :] = x_ref[j].astype(jnp.bfloat16)

        p0 = s[b0:b0 + LX, :] * m_ref[0, b0:b0 + LX, :]
        p1 = s[b0 + 1:b0 + 1 + LX, :]
        p2 = s[b0 + 2:b0 + 2 + LX, :] * m_ref[1, b0 + 2:b0 + 2 + LX, :]
        chunks.append(jnp.concatenate([p0, p1, p2], axis=1))
    x3 = jnp.concatenate(chunks, axis=0)

    u0 = jnp.dot(x3, w_ref[0], preferred_element_type=jnp.float32)
    u1 = jnp.dot(x3, w_ref[1], preferred_element_type=jnp.float32)
    u2 = jnp.dot(x3, w_ref[2], preferred_element_type=jnp.float32)

    for j in range(ipb):
        o = j * LX
        acc = (u0[o:o + L] + u1[o + wo:o + wo + L]
               + u2[o + 2 * wo:o + 2 * wo + L] + b_ref[...])
        d = 1.0 + jnp.exp(-acc)
        r = pl.reciprocal(d, approx=True)
        o_ref[j] = r * (2.0 - d * r)


def kernel(x_nchw, weight_oihw, bias):
    N, Cin, H, W = x_nchw.shape
    Cout, Cin_w, K, K2 = weight_oihw.shape
    assert Cin == Cin_w and K == K2

    Ho, Wo = H, W
    L = Ho * Wo
    R0 = 64
    ROWS = ((R0 + L + Wo + 2 + 15) // 16) * 16
    IPB = 2
    assert N % IPB == 0

    x_pix = jnp.transpose(x_nchw, (0, 2, 3, 1)).reshape(N, L, Cin)

    w3 = jnp.transpose(weight_oihw, (2, 3, 1, 0)).reshape(K, K * Cin, Cout)
    w3 = w3.astype(jnp.bfloat16)
    b2d = bias.astype(jnp.float32).reshape(1, Cout)

    r = np.arange(ROWS)
    q = r - R0
    in_data = (q >= 0) & (q < L)
    col = q % Wo
    m0 = np.where(in_data & (col == Wo - 1), 0.0, 1.0)
    m2 = np.where(in_data & (col == 0), 0.0, 1.0)
    masks = np.broadcast_to(
        np.stack([m0, m2])[:, :, None], (2, ROWS, Cin)).astype(np.float32)
    masks = jnp.asarray(masks).astype(jnp.bfloat16)

    kernel_fn = functools.partial(_conv_sig_kernel, cin=Cin, cout=Cout, k=K,
                                  ho=Ho, wo=Wo, r0=R0, rows=ROWS, ipb=IPB)

    out = pl.pallas_call(
        kernel_fn,
        out_shape=jax.ShapeDtypeStruct((N, L, Cout), x_nchw.dtype),
        grid=(N // IPB,),
        in_specs=[
            pl.BlockSpec((IPB, L, Cin), lambda n: (n, 0, 0)),
            pl.BlockSpec((K, K * Cin, Cout), lambda n: (0, 0, 0)),
            pl.BlockSpec((1, Cout), lambda n: (0, 0)),
            pl.BlockSpec((2, ROWS, Cin), lambda n: (0, 0, 0)),
        ],
        out_specs=pl.BlockSpec((IPB, L, Cout), lambda n: (n, 0, 0)),
        scratch_shapes=[pltpu.VMEM((IPB, ROWS, Cin), jnp.bfloat16)],
        compiler_params=pltpu.CompilerParams(
            dimension_semantics=("parallel",),
            vmem_limit_bytes=64 * 1024 * 1024,
        ),
    )(x_pix, w3, b2d, masks)
    return jnp.transpose(out.reshape(N, Ho, Wo, Cout), (0, 3, 1, 2))
